# baseline (speedup 1.0000x reference)
import jax
import jax.numpy as jnp
from jax.experimental import pallas as pl
from jax.experimental.pallas import tpu as pltpu

SEQ = 2048
NUM_HEADS = 16
HEAD_SIZE = 64
N_EMBED = 1024
NUM_EXPERTS = 8
TOP_K = 2
NUM_KV_HEADS = NUM_HEADS // TOP_K
KV_PROJ = NUM_KV_HEADS * HEAD_SIZE

BT = 256
BQ = 256
BKV = 256


def _dot_t(a, b):
    return jax.lax.dot_general(
        a, b, (((1,), (1,)), ((), ())), preferred_element_type=jnp.float32)


def _qkv_kernel(x_ref, xb_ref, wr_ref, br_ref, wn_ref, bn_ref, noise_ref,
                win_ref, wk_ref, wv_ref,
                q_ref, k_ref, v_ref, gm_ref):
    x = x_ref[...]
    xb = xb_ref[...]
    logits = _dot_t(x, wr_ref[...]) + br_ref[...]
    nlog = _dot_t(x, wn_ref[...]) + bn_ref[...]
    noisy = logits + noise_ref[...] * jax.nn.softplus(nlog)

    lanes = jax.lax.broadcasted_iota(jnp.int32, noisy.shape, 1)
    i0 = jnp.argmax(noisy, axis=1)
    m0 = (lanes == i0[:, None])
    v0 = jnp.max(noisy, axis=1)
    masked = jnp.where(m0, -jnp.inf, noisy)
    i1 = jnp.argmax(masked, axis=1)
    v1 = jnp.max(masked, axis=1)
    m1 = (lanes == i1[:, None])
    e1 = jnp.exp(v1 - v0)
    g0 = 1.0 / (1.0 + e1)
    g1 = e1 / (1.0 + e1)

    m0f = m0.astype(jnp.float32)
    m1f = m1.astype(jnp.float32)
    gm0 = g0[:, None] * m0f
    gm1 = g1[:, None] * m1f
    gm_ref[...] = jnp.concatenate([gm0, gm1], axis=1)

    k_ref[...] = _dot_t(xb, wk_ref[...]).astype(jnp.bfloat16)
    v_ref[...] = _dot_t(xb, wv_ref[...]).astype(jnp.bfloat16)

    q0 = jnp.zeros((x.shape[0], KV_PROJ), jnp.float32)
    q1 = jnp.zeros((x.shape[0], KV_PROJ), jnp.float32)
    for e in range(NUM_EXPERTS):
        h = _dot_t(xb, win_ref[e])
        q0 = q0 + m0f[:, e:e + 1] * h
        q1 = q1 + m1f[:, e:e + 1] * h
    q_ref[...] = jnp.concatenate([q0, q1], axis=1).astype(jnp.bfloat16)


def _attn_kernel(q_ref, k_ref, v_ref, o_ref, s_scr):
    qi = pl.program_id(1)
    R = 2 * BQ
    q = q_ref[:, 0].reshape(R, HEAD_SIZE) * jnp.bfloat16(HEAD_SIZE ** -0.5)
    rows = qi * BQ + jax.lax.broadcasted_iota(jnp.int32, (R, BKV), 0) % BQ

    def body(j, mx):
        kb = k_ref[0, pl.ds(j * BKV, BKV), :]
        s = _dot_t(q, kb)
        cols = j * BKV + jax.lax.broadcasted_iota(jnp.int32, (R, BKV), 1)
        s = jnp.where(cols <= rows, s, -jnp.inf)
        s_scr[:, pl.ds(j * BKV, BKV)] = s
        return jnp.maximum(mx, s)

    mx = jax.lax.fori_loop(
        0, qi + 1, body, jnp.full((R, BKV), -jnp.inf, jnp.float32))
    mrow = jnp.max(mx, axis=1, keepdims=True)

    def body2(j, carry):
        lsum, av = carry
        pf = jnp.exp(s_scr[:, pl.ds(j * BKV, BKV)] - mrow)
        vb = v_ref[0, pl.ds(j * BKV, BKV), :]
        av = av + jnp.dot(pf.astype(jnp.bfloat16), vb,
                          preferred_element_type=jnp.float32)
        return lsum + pf, av

    lsum, av = jax.lax.fori_loop(
        0, qi + 1, body2,
        (jnp.zeros((R, BKV), jnp.float32), jnp.zeros((R, HEAD_SIZE), jnp.float32)))
    l = jnp.sum(lsum, axis=1, keepdims=True)
    o_ref[...] = (av / l).astype(jnp.bfloat16).reshape(2, 1, BQ, HEAD_SIZE)


def _combine_kernel(ao_ref, gm_ref, wout_ref, bias_ref, y_ref):
    ao = ao_ref[...]
    gm = gm_ref[...]
    ao0 = ao[:, :KV_PROJ].astype(jnp.float32)
    ao1 = ao[:, KV_PROJ:].astype(jnp.float32)
    y = jnp.zeros((ao.shape[0], N_EMBED), jnp.float32) + bias_ref[...]
    for e in range(NUM_EXPERTS):
        c = gm[:, e:e + 1] * ao0 + gm[:, NUM_EXPERTS + e:NUM_EXPERTS + e + 1] * ao1
        y = y + _dot_t(c.astype(jnp.bfloat16), wout_ref[e])
    y_ref[...] = y


@jax.jit
def kernel(x, W_router, b_router, W_noise, b_noise, W_in, W_out, W_k, W_v,
           p_bias, noise):
    bsz, S, D = x.shape
    T = bsz * S
    xf = x.reshape(T, D)
    xb = xf.astype(jnp.bfloat16)
    win_b = W_in.astype(jnp.bfloat16)
    wout_b = W_out.astype(jnp.bfloat16)
    wk_b = W_k.astype(jnp.bfloat16)
    wv_b = W_v.astype(jnp.bfloat16)

    q, k, v, gm = pl.pallas_call(
        _qkv_kernel,
        grid=(T // BT,),
        in_specs=[
            pl.BlockSpec((BT, D), lambda i: (i, 0)),
            pl.BlockSpec((BT, D), lambda i: (i, 0)),
            pl.BlockSpec((NUM_EXPERTS, D), lambda i: (0, 0)),
            pl.BlockSpec((1, NUM_EXPERTS), lambda i: (0, 0)),
            pl.BlockSpec((NUM_EXPERTS, D), lambda i: (0, 0)),
            pl.BlockSpec((1, NUM_EXPERTS), lambda i: (0, 0)),
            pl.BlockSpec((BT, NUM_EXPERTS), lambda i: (i, 0)),
            pl.BlockSpec((NUM_EXPERTS, KV_PROJ, D), lambda i: (0, 0, 0)),
            pl.BlockSpec((KV_PROJ, D), lambda i: (0, 0)),
            pl.BlockSpec((KV_PROJ, D), lambda i: (0, 0)),
        ],
        out_specs=[
            pl.BlockSpec((BT, NUM_HEADS * HEAD_SIZE), lambda i: (i, 0)),
            pl.BlockSpec((BT, KV_PROJ), lambda i: (i, 0)),
            pl.BlockSpec((BT, KV_PROJ), lambda i: (i, 0)),
            pl.BlockSpec((BT, 2 * NUM_EXPERTS), lambda i: (i, 0)),
        ],
        out_shape=[
            jax.ShapeDtypeStruct((T, NUM_HEADS * HEAD_SIZE), jnp.bfloat16),
            jax.ShapeDtypeStruct((T, KV_PROJ), jnp.bfloat16),
            jax.ShapeDtypeStruct((T, KV_PROJ), jnp.bfloat16),
            jax.ShapeDtypeStruct((T, 2 * NUM_EXPERTS), jnp.float32),
        ],
    )(xf, xb, W_router, b_router.reshape(1, NUM_EXPERTS), W_noise,
      b_noise.reshape(1, NUM_EXPERTS), noise, win_b, wk_b, wv_b)

    qh = q.reshape(S, NUM_HEADS, HEAD_SIZE).transpose(1, 0, 2).reshape(
        TOP_K, NUM_KV_HEADS, S, HEAD_SIZE)
    kh = k.reshape(S, NUM_KV_HEADS, HEAD_SIZE).transpose(1, 0, 2)
    vh = v.reshape(S, NUM_KV_HEADS, HEAD_SIZE).transpose(1, 0, 2)

    o = pl.pallas_call(
        _attn_kernel,
        grid=(NUM_KV_HEADS, S // BQ),
        in_specs=[
            pl.BlockSpec((TOP_K, 1, BQ, HEAD_SIZE), lambda h, i: (0, h, i, 0)),
            pl.BlockSpec((1, S, HEAD_SIZE), lambda h, i: (h, 0, 0)),
            pl.BlockSpec((1, S, HEAD_SIZE), lambda h, i: (h, 0, 0)),
        ],
        out_specs=pl.BlockSpec((TOP_K, 1, BQ, HEAD_SIZE), lambda h, i: (0, h, i, 0)),
        out_shape=jax.ShapeDtypeStruct((TOP_K, NUM_KV_HEADS, S, HEAD_SIZE),
                                       jnp.bfloat16),
        scratch_shapes=[pltpu.VMEM((2 * BQ, S), jnp.float32)],
    )(qh, kh, vh)

    ao = o.reshape(NUM_HEADS, S, HEAD_SIZE).transpose(1, 0, 2).reshape(
        T, NUM_HEADS * HEAD_SIZE)

    y = pl.pallas_call(
        _combine_kernel,
        grid=(T // BT,),
        in_specs=[
            pl.BlockSpec((BT, NUM_HEADS * HEAD_SIZE), lambda i: (i, 0)),
            pl.BlockSpec((BT, 2 * NUM_EXPERTS), lambda i: (i, 0)),
            pl.BlockSpec((NUM_EXPERTS, D, KV_PROJ), lambda i: (0, 0, 0)),
            pl.BlockSpec((1, D), lambda i: (0, 0)),
        ],
        out_specs=pl.BlockSpec((BT, D), lambda i: (i, 0)),
        out_shape=jax.ShapeDtypeStruct((T, D), jnp.float32),
    )(ao, gm, wout_b, p_bias.reshape(1, D))

    return y.reshape(bsz, S, D)


# speedup vs baseline: 1.2903x; 1.2903x over previous
import jax
import jax.numpy as jnp
from jax.experimental import pallas as pl
from jax.experimental.pallas import tpu as pltpu

SEQ = 2048
NUM_HEADS = 16
HEAD_SIZE = 64
N_EMBED = 1024
NUM_EXPERTS = 8
TOP_K = 2
NUM_KV_HEADS = NUM_HEADS // TOP_K
KV_PROJ = NUM_KV_HEADS * HEAD_SIZE

BT = 256
BQ = 512
BKV = 256


def _dot_t(a, b):
    return jax.lax.dot_general(
        a, b, (((1,), (1,)), ((), ())), preferred_element_type=jnp.float32)


def _qkv_kernel(x_ref, xb_ref, wr_ref, br_ref, wn_ref, bn_ref, noise_ref,
                win_ref, wk_ref, wv_ref,
                q_ref, k_ref, v_ref, gm_ref):
    x = x_ref[...]
    xb = xb_ref[...]
    logits = _dot_t(x, wr_ref[...]) + br_ref[...]
    nlog = _dot_t(x, wn_ref[...]) + bn_ref[...]
    noisy = logits + noise_ref[...] * jax.nn.softplus(nlog)

    lanes = jax.lax.broadcasted_iota(jnp.int32, noisy.shape, 1)
    i0 = jnp.argmax(noisy, axis=1)
    m0 = (lanes == i0[:, None])
    v0 = jnp.max(noisy, axis=1)
    masked = jnp.where(m0, -jnp.inf, noisy)
    i1 = jnp.argmax(masked, axis=1)
    v1 = jnp.max(masked, axis=1)
    m1 = (lanes == i1[:, None])
    e1 = jnp.exp(v1 - v0)
    g0 = 1.0 / (1.0 + e1)
    g1 = e1 / (1.0 + e1)

    m0f = m0.astype(jnp.float32)
    m1f = m1.astype(jnp.float32)
    gm0 = g0[:, None] * m0f
    gm1 = g1[:, None] * m1f
    gm_ref[...] = jnp.concatenate([gm0, gm1], axis=1)

    k_ref[...] = _dot_t(xb, wk_ref[...]).astype(jnp.bfloat16)
    v_ref[...] = _dot_t(xb, wv_ref[...]).astype(jnp.bfloat16)

    q0 = jnp.zeros((x.shape[0], KV_PROJ), jnp.float32)
    q1 = jnp.zeros((x.shape[0], KV_PROJ), jnp.float32)
    for e in range(NUM_EXPERTS):
        h = _dot_t(xb, win_ref[e])
        q0 = q0 + m0f[:, e:e + 1] * h
        q1 = q1 + m1f[:, e:e + 1] * h
    q_ref[...] = jnp.concatenate([q0, q1], axis=1).astype(jnp.bfloat16)


def _attn_kernel(q_ref, k_ref, v_ref, o_ref, s_scr):
    qi = pl.program_id(1)
    R = 2 * BQ
    q = q_ref[:, 0].reshape(R, HEAD_SIZE) * jnp.bfloat16(HEAD_SIZE ** -0.5)
    rows = qi * BQ + jax.lax.broadcasted_iota(jnp.int32, (R, BKV), 0) % BQ

    def fill(j, _):
        s_scr[:, pl.ds(j * BKV, BKV)] = jnp.full((R, BKV), -jnp.inf, jnp.float32)
        return 0

    def body(j, _):
        kb = k_ref[0, pl.ds(j * BKV, BKV), :]
        s = _dot_t(q, kb)
        cols = j * BKV + jax.lax.broadcasted_iota(jnp.int32, (R, BKV), 1)
        s_scr[:, pl.ds(j * BKV, BKV)] = jnp.where(cols <= rows, s, -jnp.inf)
        return 0

    nvalid = (qi + 1) * (BQ // BKV)
    jax.lax.fori_loop(nvalid, SEQ // BKV, fill, 0)
    jax.lax.fori_loop(0, nvalid, body, 0)

    sf = s_scr[...]
    mrow = jnp.max(sf, axis=1, keepdims=True)
    p = jnp.exp(sf - mrow)
    l = jnp.sum(p, axis=1, keepdims=True)
    av = jnp.dot(p.astype(jnp.bfloat16), v_ref[0],
                 preferred_element_type=jnp.float32)
    o_ref[...] = (av / l).astype(jnp.bfloat16).reshape(2, 1, BQ, HEAD_SIZE)


def _combine_kernel(ao_ref, gm_ref, wout_ref, bias_ref, y_ref):
    ao = ao_ref[...]
    gm = gm_ref[...]
    ao0 = ao[:, :KV_PROJ].astype(jnp.float32)
    ao1 = ao[:, KV_PROJ:].astype(jnp.float32)
    y = jnp.zeros((ao.shape[0], N_EMBED), jnp.float32) + bias_ref[...]
    for e in range(NUM_EXPERTS):
        c = gm[:, e:e + 1] * ao0 + gm[:, NUM_EXPERTS + e:NUM_EXPERTS + e + 1] * ao1
        y = y + _dot_t(c.astype(jnp.bfloat16), wout_ref[e])
    y_ref[...] = y


@jax.jit
def kernel(x, W_router, b_router, W_noise, b_noise, W_in, W_out, W_k, W_v,
           p_bias, noise):
    bsz, S, D = x.shape
    T = bsz * S
    xf = x.reshape(T, D)
    xb = xf.astype(jnp.bfloat16)
    win_b = W_in.astype(jnp.bfloat16)
    wout_b = W_out.astype(jnp.bfloat16)
    wk_b = W_k.astype(jnp.bfloat16)
    wv_b = W_v.astype(jnp.bfloat16)

    q, k, v, gm = pl.pallas_call(
        _qkv_kernel,
        grid=(T // BT,),
        in_specs=[
            pl.BlockSpec((BT, D), lambda i: (i, 0)),
            pl.BlockSpec((BT, D), lambda i: (i, 0)),
            pl.BlockSpec((NUM_EXPERTS, D), lambda i: (0, 0)),
            pl.BlockSpec((1, NUM_EXPERTS), lambda i: (0, 0)),
            pl.BlockSpec((NUM_EXPERTS, D), lambda i: (0, 0)),
            pl.BlockSpec((1, NUM_EXPERTS), lambda i: (0, 0)),
            pl.BlockSpec((BT, NUM_EXPERTS), lambda i: (i, 0)),
            pl.BlockSpec((NUM_EXPERTS, KV_PROJ, D), lambda i: (0, 0, 0)),
            pl.BlockSpec((KV_PROJ, D), lambda i: (0, 0)),
            pl.BlockSpec((KV_PROJ, D), lambda i: (0, 0)),
        ],
        out_specs=[
            pl.BlockSpec((BT, NUM_HEADS * HEAD_SIZE), lambda i: (i, 0)),
            pl.BlockSpec((BT, KV_PROJ), lambda i: (i, 0)),
            pl.BlockSpec((BT, KV_PROJ), lambda i: (i, 0)),
            pl.BlockSpec((BT, 2 * NUM_EXPERTS), lambda i: (i, 0)),
        ],
        out_shape=[
            jax.ShapeDtypeStruct((T, NUM_HEADS * HEAD_SIZE), jnp.bfloat16),
            jax.ShapeDtypeStruct((T, KV_PROJ), jnp.bfloat16),
            jax.ShapeDtypeStruct((T, KV_PROJ), jnp.bfloat16),
            jax.ShapeDtypeStruct((T, 2 * NUM_EXPERTS), jnp.float32),
        ],
    )(xf, xb, W_router, b_router.reshape(1, NUM_EXPERTS), W_noise,
      b_noise.reshape(1, NUM_EXPERTS), noise, win_b, wk_b, wv_b)

    qh = q.reshape(S, NUM_HEADS, HEAD_SIZE).transpose(1, 0, 2).reshape(
        TOP_K, NUM_KV_HEADS, S, HEAD_SIZE)
    kh = k.reshape(S, NUM_KV_HEADS, HEAD_SIZE).transpose(1, 0, 2)
    vh = v.reshape(S, NUM_KV_HEADS, HEAD_SIZE).transpose(1, 0, 2)

    o = pl.pallas_call(
        _attn_kernel,
        grid=(NUM_KV_HEADS, S // BQ),
        in_specs=[
            pl.BlockSpec((TOP_K, 1, BQ, HEAD_SIZE), lambda h, i: (0, h, i, 0)),
            pl.BlockSpec((1, S, HEAD_SIZE), lambda h, i: (h, 0, 0)),
            pl.BlockSpec((1, S, HEAD_SIZE), lambda h, i: (h, 0, 0)),
        ],
        out_specs=pl.BlockSpec((TOP_K, 1, BQ, HEAD_SIZE), lambda h, i: (0, h, i, 0)),
        out_shape=jax.ShapeDtypeStruct((TOP_K, NUM_KV_HEADS, S, HEAD_SIZE),
                                       jnp.bfloat16),
        scratch_shapes=[pltpu.VMEM((2 * BQ, S), jnp.float32)],
    )(qh, kh, vh)

    ao = o.reshape(NUM_HEADS, S, HEAD_SIZE).transpose(1, 0, 2).reshape(
        T, NUM_HEADS * HEAD_SIZE)

    y = pl.pallas_call(
        _combine_kernel,
        grid=(T // BT,),
        in_specs=[
            pl.BlockSpec((BT, NUM_HEADS * HEAD_SIZE), lambda i: (i, 0)),
            pl.BlockSpec((BT, 2 * NUM_EXPERTS), lambda i: (i, 0)),
            pl.BlockSpec((NUM_EXPERTS, D, KV_PROJ), lambda i: (0, 0, 0)),
            pl.BlockSpec((1, D), lambda i: (0, 0)),
        ],
        out_specs=pl.BlockSpec((BT, D), lambda i: (i, 0)),
        out_shape=jax.ShapeDtypeStruct((T, D), jnp.float32),
    )(ao, gm, wout_b, p_bias.reshape(1, D))

    return y.reshape(bsz, S, D)


# speedup vs baseline: 1.3456x; 1.0429x over previous
import jax
import jax.numpy as jnp
from jax.experimental import pallas as pl
from jax.experimental.pallas import tpu as pltpu

SEQ = 2048
NUM_HEADS = 16
HEAD_SIZE = 64
N_EMBED = 1024
NUM_EXPERTS = 8
TOP_K = 2
NUM_KV_HEADS = NUM_HEADS // TOP_K
KV_PROJ = NUM_KV_HEADS * HEAD_SIZE

BT = 256
BQ = 1024
BKV = 256


def _dot_t(a, b):
    return jax.lax.dot_general(
        a, b, (((1,), (1,)), ((), ())), preferred_element_type=jnp.float32)


def _qkv_kernel(x_ref, xb_ref, wr_ref, br_ref, wn_ref, bn_ref, noise_ref,
                win_ref, wk_ref, wv_ref,
                q_ref, k_ref, v_ref, gm_ref):
    x = x_ref[...]
    xb = xb_ref[...]
    logits = _dot_t(x, wr_ref[...]) + br_ref[...]
    nlog = _dot_t(x, wn_ref[...]) + bn_ref[...]
    noisy = logits + noise_ref[...] * jax.nn.softplus(nlog)

    lanes = jax.lax.broadcasted_iota(jnp.int32, noisy.shape, 1)
    i0 = jnp.argmax(noisy, axis=1)
    m0 = (lanes == i0[:, None])
    v0 = jnp.max(noisy, axis=1)
    masked = jnp.where(m0, -jnp.inf, noisy)
    i1 = jnp.argmax(masked, axis=1)
    v1 = jnp.max(masked, axis=1)
    m1 = (lanes == i1[:, None])
    e1 = jnp.exp(v1 - v0)
    g0 = 1.0 / (1.0 + e1)
    g1 = e1 / (1.0 + e1)

    m0f = m0.astype(jnp.float32)
    m1f = m1.astype(jnp.float32)
    gm0 = g0[:, None] * m0f
    gm1 = g1[:, None] * m1f
    gm_ref[...] = jnp.concatenate([gm0, gm1], axis=1)

    k_ref[...] = _dot_t(xb, wk_ref[...]).astype(jnp.bfloat16)
    v_ref[...] = _dot_t(xb, wv_ref[...]).astype(jnp.bfloat16)

    q0 = jnp.zeros((x.shape[0], KV_PROJ), jnp.float32)
    q1 = jnp.zeros((x.shape[0], KV_PROJ), jnp.float32)
    for e in range(NUM_EXPERTS):
        h = _dot_t(xb, win_ref[e])
        q0 = q0 + m0f[:, e:e + 1] * h
        q1 = q1 + m1f[:, e:e + 1] * h
    q_ref[...] = jnp.concatenate([q0, q1], axis=1).astype(jnp.bfloat16)


def _attn_kernel(q_ref, k_ref, v_ref, o_ref, s_scr):
    qi = pl.program_id(1)
    R = 2 * BQ
    q = q_ref[:, 0].reshape(R, HEAD_SIZE) * jnp.bfloat16(HEAD_SIZE ** -0.5)
    rows = qi * BQ + jax.lax.broadcasted_iota(jnp.int32, (R, BKV), 0) % BQ

    def fill(j, _):
        s_scr[:, pl.ds(j * BKV, BKV)] = jnp.full((R, BKV), -jnp.inf, jnp.float32)
        return 0

    def body(j, _):
        kb = k_ref[0, pl.ds(j * BKV, BKV), :]
        s = _dot_t(q, kb)
        cols = j * BKV + jax.lax.broadcasted_iota(jnp.int32, (R, BKV), 1)
        s_scr[:, pl.ds(j * BKV, BKV)] = jnp.where(cols <= rows, s, -jnp.inf)
        return 0

    nvalid = (qi + 1) * (BQ // BKV)
    jax.lax.fori_loop(nvalid, SEQ // BKV, fill, 0)
    jax.lax.fori_loop(0, nvalid, body, 0)

    sf = s_scr[...]
    mrow = jnp.max(sf, axis=1, keepdims=True)
    p = jnp.exp(sf - mrow)
    l = jnp.sum(p, axis=1, keepdims=True)
    av = jnp.dot(p.astype(jnp.bfloat16), v_ref[0],
                 preferred_element_type=jnp.float32)
    o_ref[...] = (av / l).astype(jnp.bfloat16).reshape(2, 1, BQ, HEAD_SIZE)


def _combine_kernel(ao_ref, gm_ref, wout_ref, bias_ref, y_ref):
    ao = ao_ref[...]
    gm = gm_ref[...]
    ao0 = ao[:, :KV_PROJ].astype(jnp.float32)
    ao1 = ao[:, KV_PROJ:].astype(jnp.float32)
    y = jnp.zeros((ao.shape[0], N_EMBED), jnp.float32) + bias_ref[...]
    for e in range(NUM_EXPERTS):
        c = gm[:, e:e + 1] * ao0 + gm[:, NUM_EXPERTS + e:NUM_EXPERTS + e + 1] * ao1
        y = y + _dot_t(c.astype(jnp.bfloat16), wout_ref[e])
    y_ref[...] = y


@jax.jit
def kernel(x, W_router, b_router, W_noise, b_noise, W_in, W_out, W_k, W_v,
           p_bias, noise):
    bsz, S, D = x.shape
    T = bsz * S
    xf = x.reshape(T, D)
    xb = xf.astype(jnp.bfloat16)
    win_b = W_in.astype(jnp.bfloat16)
    wout_b = W_out.astype(jnp.bfloat16)
    wk_b = W_k.astype(jnp.bfloat16)
    wv_b = W_v.astype(jnp.bfloat16)

    q, k, v, gm = pl.pallas_call(
        _qkv_kernel,
        grid=(T // BT,),
        in_specs=[
            pl.BlockSpec((BT, D), lambda i: (i, 0)),
            pl.BlockSpec((BT, D), lambda i: (i, 0)),
            pl.BlockSpec((NUM_EXPERTS, D), lambda i: (0, 0)),
            pl.BlockSpec((1, NUM_EXPERTS), lambda i: (0, 0)),
            pl.BlockSpec((NUM_EXPERTS, D), lambda i: (0, 0)),
            pl.BlockSpec((1, NUM_EXPERTS), lambda i: (0, 0)),
            pl.BlockSpec((BT, NUM_EXPERTS), lambda i: (i, 0)),
            pl.BlockSpec((NUM_EXPERTS, KV_PROJ, D), lambda i: (0, 0, 0)),
            pl.BlockSpec((KV_PROJ, D), lambda i: (0, 0)),
            pl.BlockSpec((KV_PROJ, D), lambda i: (0, 0)),
        ],
        out_specs=[
            pl.BlockSpec((BT, NUM_HEADS * HEAD_SIZE), lambda i: (i, 0)),
            pl.BlockSpec((BT, KV_PROJ), lambda i: (i, 0)),
            pl.BlockSpec((BT, KV_PROJ), lambda i: (i, 0)),
            pl.BlockSpec((BT, 2 * NUM_EXPERTS), lambda i: (i, 0)),
        ],
        out_shape=[
            jax.ShapeDtypeStruct((T, NUM_HEADS * HEAD_SIZE), jnp.bfloat16),
            jax.ShapeDtypeStruct((T, KV_PROJ), jnp.bfloat16),
            jax.ShapeDtypeStruct((T, KV_PROJ), jnp.bfloat16),
            jax.ShapeDtypeStruct((T, 2 * NUM_EXPERTS), jnp.float32),
        ],
    )(xf, xb, W_router, b_router.reshape(1, NUM_EXPERTS), W_noise,
      b_noise.reshape(1, NUM_EXPERTS), noise, win_b, wk_b, wv_b)

    qh = q.reshape(S, NUM_HEADS, HEAD_SIZE).transpose(1, 0, 2).reshape(
        TOP_K, NUM_KV_HEADS, S, HEAD_SIZE)
    kh = k.reshape(S, NUM_KV_HEADS, HEAD_SIZE).transpose(1, 0, 2)
    vh = v.reshape(S, NUM_KV_HEADS, HEAD_SIZE).transpose(1, 0, 2)

    o = pl.pallas_call(
        _attn_kernel,
        grid=(NUM_KV_HEADS, S // BQ),
        in_specs=[
            pl.BlockSpec((TOP_K, 1, BQ, HEAD_SIZE), lambda h, i: (0, h, i, 0)),
            pl.BlockSpec((1, S, HEAD_SIZE), lambda h, i: (h, 0, 0)),
            pl.BlockSpec((1, S, HEAD_SIZE), lambda h, i: (h, 0, 0)),
        ],
        out_specs=pl.BlockSpec((TOP_K, 1, BQ, HEAD_SIZE), lambda h, i: (0, h, i, 0)),
        out_shape=jax.ShapeDtypeStruct((TOP_K, NUM_KV_HEADS, S, HEAD_SIZE),
                                       jnp.bfloat16),
        scratch_shapes=[pltpu.VMEM((2 * BQ, S), jnp.float32)],
    )(qh, kh, vh)

    ao = o.reshape(NUM_HEADS, S, HEAD_SIZE).transpose(1, 0, 2).reshape(
        T, NUM_HEADS * HEAD_SIZE)

    y = pl.pallas_call(
        _combine_kernel,
        grid=(T // BT,),
        in_specs=[
            pl.BlockSpec((BT, NUM_HEADS * HEAD_SIZE), lambda i: (i, 0)),
            pl.BlockSpec((BT, 2 * NUM_EXPERTS), lambda i: (i, 0)),
            pl.BlockSpec((NUM_EXPERTS, D, KV_PROJ), lambda i: (0, 0, 0)),
            pl.BlockSpec((1, D), lambda i: (0, 0)),
        ],
        out_specs=pl.BlockSpec((BT, D), lambda i: (i, 0)),
        out_shape=jax.ShapeDtypeStruct((T, D), jnp.float32),
    )(ao, gm, wout_b, p_bias.reshape(1, D))

    return y.reshape(bsz, S, D)


# speedup vs baseline: 1.4061x; 1.0450x over previous
import jax
import jax.numpy as jnp
from jax.experimental import pallas as pl
from jax.experimental.pallas import tpu as pltpu

SEQ = 2048
NUM_HEADS = 16
HEAD_SIZE = 64
N_EMBED = 1024
NUM_EXPERTS = 8
TOP_K = 2
NUM_KV_HEADS = NUM_HEADS // TOP_K
KV_PROJ = NUM_KV_HEADS * HEAD_SIZE

BT = 256
BQ = 1024
BKV = 512


def _dot_t(a, b):
    return jax.lax.dot_general(
        a, b, (((1,), (1,)), ((), ())), preferred_element_type=jnp.float32)


def _qkv_kernel(x_ref, xb_ref, wr_ref, br_ref, wn_ref, bn_ref, noise_ref,
                win_ref, wk_ref, wv_ref,
                q_ref, k_ref, v_ref, gm_ref):
    x = x_ref[...]
    xb = xb_ref[...]
    logits = _dot_t(x, wr_ref[...]) + br_ref[...]
    nlog = _dot_t(x, wn_ref[...]) + bn_ref[...]
    noisy = logits + noise_ref[...] * jax.nn.softplus(nlog)

    lanes = jax.lax.broadcasted_iota(jnp.int32, noisy.shape, 1)
    i0 = jnp.argmax(noisy, axis=1)
    m0 = (lanes == i0[:, None])
    v0 = jnp.max(noisy, axis=1)
    masked = jnp.where(m0, -jnp.inf, noisy)
    i1 = jnp.argmax(masked, axis=1)
    v1 = jnp.max(masked, axis=1)
    m1 = (lanes == i1[:, None])
    e1 = jnp.exp(v1 - v0)
    g0 = 1.0 / (1.0 + e1)
    g1 = e1 / (1.0 + e1)

    m0f = m0.astype(jnp.float32)
    m1f = m1.astype(jnp.float32)
    gm0 = g0[:, None] * m0f
    gm1 = g1[:, None] * m1f
    gm_ref[...] = jnp.concatenate([gm0, gm1], axis=1)

    k_ref[...] = _dot_t(xb, wk_ref[...]).astype(jnp.bfloat16)
    v_ref[...] = _dot_t(xb, wv_ref[...]).astype(jnp.bfloat16)

    q0 = jnp.zeros((x.shape[0], KV_PROJ), jnp.float32)
    q1 = jnp.zeros((x.shape[0], KV_PROJ), jnp.float32)
    for e in range(NUM_EXPERTS):
        h = _dot_t(xb, win_ref[e])
        q0 = q0 + m0f[:, e:e + 1] * h
        q1 = q1 + m1f[:, e:e + 1] * h
    q_ref[...] = jnp.concatenate([q0, q1], axis=1).astype(jnp.bfloat16)


def _attn_kernel(q_ref, k_ref, v_ref, o_ref, s_scr):
    qi = pl.program_id(1)
    R = 2 * BQ
    q = q_ref[:, 0].reshape(R, HEAD_SIZE) * jnp.bfloat16(HEAD_SIZE ** -0.5)
    rows = qi * BQ + jax.lax.broadcasted_iota(jnp.int32, (R, BKV), 0) % BQ

    def fill(j, _):
        s_scr[:, pl.ds(j * BKV, BKV)] = jnp.full((R, BKV), -jnp.inf, jnp.float32)
        return 0

    def body(j, _):
        kb = k_ref[0, pl.ds(j * BKV, BKV), :]
        s = _dot_t(q, kb)
        cols = j * BKV + jax.lax.broadcasted_iota(jnp.int32, (R, BKV), 1)
        s_scr[:, pl.ds(j * BKV, BKV)] = jnp.where(cols <= rows, s, -jnp.inf)
        return 0

    nvalid = (qi + 1) * (BQ // BKV)
    jax.lax.fori_loop(nvalid, SEQ // BKV, fill, 0)
    jax.lax.fori_loop(0, nvalid, body, 0)

    sf = s_scr[...]
    mrow = jnp.max(sf, axis=1, keepdims=True)
    p = jnp.exp(sf - mrow)
    l = jnp.sum(p, axis=1, keepdims=True)
    av = jnp.dot(p.astype(jnp.bfloat16), v_ref[0],
                 preferred_element_type=jnp.float32)
    o_ref[...] = (av / l).astype(jnp.bfloat16).reshape(2, 1, BQ, HEAD_SIZE)


def _combine_kernel(ao_ref, gm_ref, wout_ref, bias_ref, y_ref):
    ao = ao_ref[...]
    gm = gm_ref[...]
    ao0 = ao[:, :KV_PROJ].astype(jnp.float32)
    ao1 = ao[:, KV_PROJ:].astype(jnp.float32)
    y = jnp.zeros((ao.shape[0], N_EMBED), jnp.float32) + bias_ref[...]
    for e in range(NUM_EXPERTS):
        c = gm[:, e:e + 1] * ao0 + gm[:, NUM_EXPERTS + e:NUM_EXPERTS + e + 1] * ao1
        y = y + _dot_t(c.astype(jnp.bfloat16), wout_ref[e])
    y_ref[...] = y


@jax.jit
def kernel(x, W_router, b_router, W_noise, b_noise, W_in, W_out, W_k, W_v,
           p_bias, noise):
    bsz, S, D = x.shape
    T = bsz * S
    xf = x.reshape(T, D)
    xb = xf.astype(jnp.bfloat16)
    win_b = W_in.astype(jnp.bfloat16)
    wout_b = W_out.astype(jnp.bfloat16)
    wk_b = W_k.astype(jnp.bfloat16)
    wv_b = W_v.astype(jnp.bfloat16)

    q, k, v, gm = pl.pallas_call(
        _qkv_kernel,
        grid=(T // BT,),
        in_specs=[
            pl.BlockSpec((BT, D), lambda i: (i, 0)),
            pl.BlockSpec((BT, D), lambda i: (i, 0)),
            pl.BlockSpec((NUM_EXPERTS, D), lambda i: (0, 0)),
            pl.BlockSpec((1, NUM_EXPERTS), lambda i: (0, 0)),
            pl.BlockSpec((NUM_EXPERTS, D), lambda i: (0, 0)),
            pl.BlockSpec((1, NUM_EXPERTS), lambda i: (0, 0)),
            pl.BlockSpec((BT, NUM_EXPERTS), lambda i: (i, 0)),
            pl.BlockSpec((NUM_EXPERTS, KV_PROJ, D), lambda i: (0, 0, 0)),
            pl.BlockSpec((KV_PROJ, D), lambda i: (0, 0)),
            pl.BlockSpec((KV_PROJ, D), lambda i: (0, 0)),
        ],
        out_specs=[
            pl.BlockSpec((BT, NUM_HEADS * HEAD_SIZE), lambda i: (i, 0)),
            pl.BlockSpec((BT, KV_PROJ), lambda i: (i, 0)),
            pl.BlockSpec((BT, KV_PROJ), lambda i: (i, 0)),
            pl.BlockSpec((BT, 2 * NUM_EXPERTS), lambda i: (i, 0)),
        ],
        out_shape=[
            jax.ShapeDtypeStruct((T, NUM_HEADS * HEAD_SIZE), jnp.bfloat16),
            jax.ShapeDtypeStruct((T, KV_PROJ), jnp.bfloat16),
            jax.ShapeDtypeStruct((T, KV_PROJ), jnp.bfloat16),
            jax.ShapeDtypeStruct((T, 2 * NUM_EXPERTS), jnp.float32),
        ],
    )(xf, xb, W_router, b_router.reshape(1, NUM_EXPERTS), W_noise,
      b_noise.reshape(1, NUM_EXPERTS), noise, win_b, wk_b, wv_b)

    qh = q.reshape(S, NUM_HEADS, HEAD_SIZE).transpose(1, 0, 2).reshape(
        TOP_K, NUM_KV_HEADS, S, HEAD_SIZE)
    kh = k.reshape(S, NUM_KV_HEADS, HEAD_SIZE).transpose(1, 0, 2)
    vh = v.reshape(S, NUM_KV_HEADS, HEAD_SIZE).transpose(1, 0, 2)

    o = pl.pallas_call(
        _attn_kernel,
        grid=(NUM_KV_HEADS, S // BQ),
        in_specs=[
            pl.BlockSpec((TOP_K, 1, BQ, HEAD_SIZE), lambda h, i: (0, h, i, 0)),
            pl.BlockSpec((1, S, HEAD_SIZE), lambda h, i: (h, 0, 0)),
            pl.BlockSpec((1, S, HEAD_SIZE), lambda h, i: (h, 0, 0)),
        ],
        out_specs=pl.BlockSpec((TOP_K, 1, BQ, HEAD_SIZE), lambda h, i: (0, h, i, 0)),
        out_shape=jax.ShapeDtypeStruct((TOP_K, NUM_KV_HEADS, S, HEAD_SIZE),
                                       jnp.bfloat16),
        scratch_shapes=[pltpu.VMEM((2 * BQ, S), jnp.float32)],
    )(qh, kh, vh)

    ao = o.reshape(NUM_HEADS, S, HEAD_SIZE).transpose(1, 0, 2).reshape(
        T, NUM_HEADS * HEAD_SIZE)

    y = pl.pallas_call(
        _combine_kernel,
        grid=(T // BT,),
        in_specs=[
            pl.BlockSpec((BT, NUM_HEADS * HEAD_SIZE), lambda i: (i, 0)),
            pl.BlockSpec((BT, 2 * NUM_EXPERTS), lambda i: (i, 0)),
            pl.BlockSpec((NUM_EXPERTS, D, KV_PROJ), lambda i: (0, 0, 0)),
            pl.BlockSpec((1, D), lambda i: (0, 0)),
        ],
        out_specs=pl.BlockSpec((BT, D), lambda i: (i, 0)),
        out_shape=jax.ShapeDtypeStruct((T, D), jnp.float32),
    )(ao, gm, wout_b, p_bias.reshape(1, D))

    return y.reshape(bsz, S, D)


# speedup vs baseline: 1.4335x; 1.0195x over previous
import jax
import jax.numpy as jnp
from jax.experimental import pallas as pl
from jax.experimental.pallas import tpu as pltpu

SEQ = 2048
NUM_HEADS = 16
HEAD_SIZE = 64
N_EMBED = 1024
NUM_EXPERTS = 8
TOP_K = 2
NUM_KV_HEADS = NUM_HEADS // TOP_K
KV_PROJ = NUM_KV_HEADS * HEAD_SIZE

BT = 256
BQ = 1024
BKV = 1024


def _dot_t(a, b):
    return jax.lax.dot_general(
        a, b, (((1,), (1,)), ((), ())), preferred_element_type=jnp.float32)


def _qkv_kernel(x_ref, xb_ref, wr_ref, br_ref, wn_ref, bn_ref, noise_ref,
                win_ref, wk_ref, wv_ref,
                q_ref, k_ref, v_ref, gm_ref):
    x = x_ref[...]
    xb = xb_ref[...]
    logits = _dot_t(x, wr_ref[...]) + br_ref[...]
    nlog = _dot_t(x, wn_ref[...]) + bn_ref[...]
    noisy = logits + noise_ref[...] * jax.nn.softplus(nlog)

    lanes = jax.lax.broadcasted_iota(jnp.int32, noisy.shape, 1)
    i0 = jnp.argmax(noisy, axis=1)
    m0 = (lanes == i0[:, None])
    v0 = jnp.max(noisy, axis=1)
    masked = jnp.where(m0, -jnp.inf, noisy)
    i1 = jnp.argmax(masked, axis=1)
    v1 = jnp.max(masked, axis=1)
    m1 = (lanes == i1[:, None])
    e1 = jnp.exp(v1 - v0)
    g0 = 1.0 / (1.0 + e1)
    g1 = e1 / (1.0 + e1)

    m0f = m0.astype(jnp.float32)
    m1f = m1.astype(jnp.float32)
    gm0 = g0[:, None] * m0f
    gm1 = g1[:, None] * m1f
    gm_ref[...] = jnp.concatenate([gm0, gm1], axis=1)

    k_ref[...] = _dot_t(xb, wk_ref[...]).astype(jnp.bfloat16)
    v_ref[...] = _dot_t(xb, wv_ref[...]).astype(jnp.bfloat16)

    q0 = jnp.zeros((x.shape[0], KV_PROJ), jnp.float32)
    q1 = jnp.zeros((x.shape[0], KV_PROJ), jnp.float32)
    for e in range(NUM_EXPERTS):
        h = _dot_t(xb, win_ref[e])
        q0 = q0 + m0f[:, e:e + 1] * h
        q1 = q1 + m1f[:, e:e + 1] * h
    q_ref[...] = jnp.concatenate([q0, q1], axis=1).astype(jnp.bfloat16)


def _attn_kernel(q_ref, k_ref, v_ref, o_ref, s_scr):
    qi = pl.program_id(1)
    R = 2 * BQ
    q = q_ref[:, 0].reshape(R, HEAD_SIZE) * jnp.bfloat16(HEAD_SIZE ** -0.5)
    rows = qi * BQ + jax.lax.broadcasted_iota(jnp.int32, (R, BKV), 0) % BQ

    def fill(j, _):
        s_scr[:, pl.ds(j * BKV, BKV)] = jnp.full((R, BKV), -jnp.inf, jnp.float32)
        return 0

    def body(j, _):
        kb = k_ref[0, pl.ds(j * BKV, BKV), :]
        s = _dot_t(q, kb)
        cols = j * BKV + jax.lax.broadcasted_iota(jnp.int32, (R, BKV), 1)
        s_scr[:, pl.ds(j * BKV, BKV)] = jnp.where(cols <= rows, s, -jnp.inf)
        return 0

    nvalid = (qi + 1) * (BQ // BKV)
    jax.lax.fori_loop(nvalid, SEQ // BKV, fill, 0)
    jax.lax.fori_loop(0, nvalid, body, 0)

    sf = s_scr[...]
    mrow = jnp.max(sf, axis=1, keepdims=True)
    p = jnp.exp(sf - mrow)
    l = jnp.sum(p, axis=1, keepdims=True)
    av = jnp.dot(p.astype(jnp.bfloat16), v_ref[0],
                 preferred_element_type=jnp.float32)
    o_ref[...] = (av / l).astype(jnp.bfloat16).reshape(2, 1, BQ, HEAD_SIZE)


def _combine_kernel(ao_ref, gm_ref, wout_ref, bias_ref, y_ref):
    ao = ao_ref[...]
    gm = gm_ref[...]
    ao0 = ao[:, :KV_PROJ].astype(jnp.float32)
    ao1 = ao[:, KV_PROJ:].astype(jnp.float32)
    y = jnp.zeros((ao.shape[0], N_EMBED), jnp.float32) + bias_ref[...]
    for e in range(NUM_EXPERTS):
        c = gm[:, e:e + 1] * ao0 + gm[:, NUM_EXPERTS + e:NUM_EXPERTS + e + 1] * ao1
        y = y + _dot_t(c.astype(jnp.bfloat16), wout_ref[e])
    y_ref[...] = y


@jax.jit
def kernel(x, W_router, b_router, W_noise, b_noise, W_in, W_out, W_k, W_v,
           p_bias, noise):
    bsz, S, D = x.shape
    T = bsz * S
    xf = x.reshape(T, D)
    xb = xf.astype(jnp.bfloat16)
    win_b = W_in.astype(jnp.bfloat16)
    wout_b = W_out.astype(jnp.bfloat16)
    wk_b = W_k.astype(jnp.bfloat16)
    wv_b = W_v.astype(jnp.bfloat16)

    q, k, v, gm = pl.pallas_call(
        _qkv_kernel,
        grid=(T // BT,),
        in_specs=[
            pl.BlockSpec((BT, D), lambda i: (i, 0)),
            pl.BlockSpec((BT, D), lambda i: (i, 0)),
            pl.BlockSpec((NUM_EXPERTS, D), lambda i: (0, 0)),
            pl.BlockSpec((1, NUM_EXPERTS), lambda i: (0, 0)),
            pl.BlockSpec((NUM_EXPERTS, D), lambda i: (0, 0)),
            pl.BlockSpec((1, NUM_EXPERTS), lambda i: (0, 0)),
            pl.BlockSpec((BT, NUM_EXPERTS), lambda i: (i, 0)),
            pl.BlockSpec((NUM_EXPERTS, KV_PROJ, D), lambda i: (0, 0, 0)),
            pl.BlockSpec((KV_PROJ, D), lambda i: (0, 0)),
            pl.BlockSpec((KV_PROJ, D), lambda i: (0, 0)),
        ],
        out_specs=[
            pl.BlockSpec((BT, NUM_HEADS * HEAD_SIZE), lambda i: (i, 0)),
            pl.BlockSpec((BT, KV_PROJ), lambda i: (i, 0)),
            pl.BlockSpec((BT, KV_PROJ), lambda i: (i, 0)),
            pl.BlockSpec((BT, 2 * NUM_EXPERTS), lambda i: (i, 0)),
        ],
        out_shape=[
            jax.ShapeDtypeStruct((T, NUM_HEADS * HEAD_SIZE), jnp.bfloat16),
            jax.ShapeDtypeStruct((T, KV_PROJ), jnp.bfloat16),
            jax.ShapeDtypeStruct((T, KV_PROJ), jnp.bfloat16),
            jax.ShapeDtypeStruct((T, 2 * NUM_EXPERTS), jnp.float32),
        ],
    )(xf, xb, W_router, b_router.reshape(1, NUM_EXPERTS), W_noise,
      b_noise.reshape(1, NUM_EXPERTS), noise, win_b, wk_b, wv_b)

    qh = q.reshape(S, NUM_HEADS, HEAD_SIZE).transpose(1, 0, 2).reshape(
        TOP_K, NUM_KV_HEADS, S, HEAD_SIZE)
    kh = k.reshape(S, NUM_KV_HEADS, HEAD_SIZE).transpose(1, 0, 2)
    vh = v.reshape(S, NUM_KV_HEADS, HEAD_SIZE).transpose(1, 0, 2)

    o = pl.pallas_call(
        _attn_kernel,
        grid=(NUM_KV_HEADS, S // BQ),
        in_specs=[
            pl.BlockSpec((TOP_K, 1, BQ, HEAD_SIZE), lambda h, i: (0, h, i, 0)),
            pl.BlockSpec((1, S, HEAD_SIZE), lambda h, i: (h, 0, 0)),
            pl.BlockSpec((1, S, HEAD_SIZE), lambda h, i: (h, 0, 0)),
        ],
        out_specs=pl.BlockSpec((TOP_K, 1, BQ, HEAD_SIZE), lambda h, i: (0, h, i, 0)),
        out_shape=jax.ShapeDtypeStruct((TOP_K, NUM_KV_HEADS, S, HEAD_SIZE),
                                       jnp.bfloat16),
        scratch_shapes=[pltpu.VMEM((2 * BQ, S), jnp.float32)],
    )(qh, kh, vh)

    ao = o.reshape(NUM_HEADS, S, HEAD_SIZE).transpose(1, 0, 2).reshape(
        T, NUM_HEADS * HEAD_SIZE)

    y = pl.pallas_call(
        _combine_kernel,
        grid=(T // BT,),
        in_specs=[
            pl.BlockSpec((BT, NUM_HEADS * HEAD_SIZE), lambda i: (i, 0)),
            pl.BlockSpec((BT, 2 * NUM_EXPERTS), lambda i: (i, 0)),
            pl.BlockSpec((NUM_EXPERTS, D, KV_PROJ), lambda i: (0, 0, 0)),
            pl.BlockSpec((1, D), lambda i: (0, 0)),
        ],
        out_specs=pl.BlockSpec((BT, D), lambda i: (i, 0)),
        out_shape=jax.ShapeDtypeStruct((T, D), jnp.float32),
    )(ao, gm, wout_b, p_bias.reshape(1, D))

    return y.reshape(bsz, S, D)


# speedup vs baseline: 1.5872x; 1.1072x over previous
import jax
import jax.numpy as jnp
from jax.experimental import pallas as pl
from jax.experimental.pallas import tpu as pltpu

SEQ = 2048
NUM_HEADS = 16
HEAD_SIZE = 64
N_EMBED = 1024
NUM_EXPERTS = 8
TOP_K = 2
NUM_KV_HEADS = NUM_HEADS // TOP_K
KV_PROJ = NUM_KV_HEADS * HEAD_SIZE

BT = 256
BQ = 1024
BKV = 1024


def _dot_t(a, b):
    return jax.lax.dot_general(
        a, b, (((1,), (1,)), ((), ())), preferred_element_type=jnp.float32)


def _qkv_kernel(x_ref, xb_ref, wr_ref, br_ref, wn_ref, bn_ref, noise_ref,
                win_ref, wk_ref, wv_ref,
                q0_ref, q1_ref, k_ref, v_ref, gm_ref):
    x = x_ref[...]
    xb = xb_ref[...]
    logits = _dot_t(x, wr_ref[...]) + br_ref[...]
    nlog = _dot_t(x, wn_ref[...]) + bn_ref[...]
    noisy = logits + noise_ref[...] * jax.nn.softplus(nlog)

    lanes = jax.lax.broadcasted_iota(jnp.int32, noisy.shape, 1)
    i0 = jnp.argmax(noisy, axis=1)
    m0 = (lanes == i0[:, None])
    v0 = jnp.max(noisy, axis=1)
    masked = jnp.where(m0, -jnp.inf, noisy)
    i1 = jnp.argmax(masked, axis=1)
    v1 = jnp.max(masked, axis=1)
    m1 = (lanes == i1[:, None])
    e1 = jnp.exp(v1 - v0)
    g0 = 1.0 / (1.0 + e1)
    g1 = e1 / (1.0 + e1)

    m0f = m0.astype(jnp.float32)
    m1f = m1.astype(jnp.float32)
    gm0 = g0[:, None] * m0f
    gm1 = g1[:, None] * m1f
    gm_ref[...] = jnp.concatenate([gm0, gm1], axis=1)

    k_ref[...] = _dot_t(xb, wk_ref[...]).astype(jnp.bfloat16).reshape(
        -1, NUM_KV_HEADS, HEAD_SIZE).transpose(1, 0, 2)
    v_ref[...] = _dot_t(xb, wv_ref[...]).astype(jnp.bfloat16).reshape(
        -1, NUM_KV_HEADS, HEAD_SIZE).transpose(1, 0, 2)

    q0 = jnp.zeros((x.shape[0], KV_PROJ), jnp.float32)
    q1 = jnp.zeros((x.shape[0], KV_PROJ), jnp.float32)
    for e in range(NUM_EXPERTS):
        h = _dot_t(xb, win_ref[e])
        q0 = q0 + m0f[:, e:e + 1] * h
        q1 = q1 + m1f[:, e:e + 1] * h
    q0_ref[...] = q0.astype(jnp.bfloat16).reshape(
        -1, NUM_KV_HEADS, HEAD_SIZE).transpose(1, 0, 2)
    q1_ref[...] = q1.astype(jnp.bfloat16).reshape(
        -1, NUM_KV_HEADS, HEAD_SIZE).transpose(1, 0, 2)


def _attn_kernel(q0_ref, q1_ref, k_ref, v_ref, o_ref, s_scr):
    qi = pl.program_id(1)
    R = 2 * BQ
    q = jnp.concatenate([q0_ref[0], q1_ref[0]], axis=0)
    q = q * jnp.bfloat16(HEAD_SIZE ** -0.5)
    rows = qi * BQ + jax.lax.broadcasted_iota(jnp.int32, (R, BKV), 0) % BQ

    def fill(j, _):
        s_scr[:, pl.ds(j * BKV, BKV)] = jnp.full((R, BKV), -jnp.inf, jnp.float32)
        return 0

    def body(j, _):
        kb = k_ref[0, pl.ds(j * BKV, BKV), :]
        s = _dot_t(q, kb)
        cols = j * BKV + jax.lax.broadcasted_iota(jnp.int32, (R, BKV), 1)
        s_scr[:, pl.ds(j * BKV, BKV)] = jnp.where(cols <= rows, s, -jnp.inf)
        return 0

    nvalid = (qi + 1) * (BQ // BKV)
    jax.lax.fori_loop(nvalid, SEQ // BKV, fill, 0)
    jax.lax.fori_loop(0, nvalid, body, 0)

    sf = s_scr[...]
    mrow = jnp.max(sf, axis=1, keepdims=True)
    p = jnp.exp(sf - mrow)
    l = jnp.sum(p, axis=1, keepdims=True)
    av = jnp.dot(p.astype(jnp.bfloat16), v_ref[0],
                 preferred_element_type=jnp.float32)
    o_ref[...] = (av / l).astype(jnp.bfloat16).reshape(2, 1, BQ, HEAD_SIZE)


def _combine_kernel(ao0_ref, ao1_ref, gm_ref, wout_ref, bias_ref, y_ref):
    gm = gm_ref[...]
    bt = ao0_ref.shape[2]
    ao0 = ao0_ref[0].transpose(1, 0, 2).reshape(bt, KV_PROJ).astype(jnp.float32)
    ao1 = ao1_ref[0].transpose(1, 0, 2).reshape(bt, KV_PROJ).astype(jnp.float32)
    y = jnp.zeros((ao0.shape[0], N_EMBED), jnp.float32) + bias_ref[...]
    for e in range(NUM_EXPERTS):
        c = gm[:, e:e + 1] * ao0 + gm[:, NUM_EXPERTS + e:NUM_EXPERTS + e + 1] * ao1
        y = y + _dot_t(c.astype(jnp.bfloat16), wout_ref[e])
    y_ref[...] = y


@jax.jit
def kernel(x, W_router, b_router, W_noise, b_noise, W_in, W_out, W_k, W_v,
           p_bias, noise):
    bsz, S, D = x.shape
    T = bsz * S
    xf = x.reshape(T, D)
    xb = xf.astype(jnp.bfloat16)
    win_b = W_in.astype(jnp.bfloat16)
    wout_b = W_out.astype(jnp.bfloat16)
    wk_b = W_k.astype(jnp.bfloat16)
    wv_b = W_v.astype(jnp.bfloat16)

    q0, q1, k, v, gm = pl.pallas_call(
        _qkv_kernel,
        grid=(T // BT,),
        in_specs=[
            pl.BlockSpec((BT, D), lambda i: (i, 0)),
            pl.BlockSpec((BT, D), lambda i: (i, 0)),
            pl.BlockSpec((NUM_EXPERTS, D), lambda i: (0, 0)),
            pl.BlockSpec((1, NUM_EXPERTS), lambda i: (0, 0)),
            pl.BlockSpec((NUM_EXPERTS, D), lambda i: (0, 0)),
            pl.BlockSpec((1, NUM_EXPERTS), lambda i: (0, 0)),
            pl.BlockSpec((BT, NUM_EXPERTS), lambda i: (i, 0)),
            pl.BlockSpec((NUM_EXPERTS, KV_PROJ, D), lambda i: (0, 0, 0)),
            pl.BlockSpec((KV_PROJ, D), lambda i: (0, 0)),
            pl.BlockSpec((KV_PROJ, D), lambda i: (0, 0)),
        ],
        out_specs=[
            pl.BlockSpec((NUM_KV_HEADS, BT, HEAD_SIZE), lambda i: (0, i, 0)),
            pl.BlockSpec((NUM_KV_HEADS, BT, HEAD_SIZE), lambda i: (0, i, 0)),
            pl.BlockSpec((NUM_KV_HEADS, BT, HEAD_SIZE), lambda i: (0, i, 0)),
            pl.BlockSpec((NUM_KV_HEADS, BT, HEAD_SIZE), lambda i: (0, i, 0)),
            pl.BlockSpec((BT, 2 * NUM_EXPERTS), lambda i: (i, 0)),
        ],
        out_shape=[
            jax.ShapeDtypeStruct((NUM_KV_HEADS, S, HEAD_SIZE), jnp.bfloat16),
            jax.ShapeDtypeStruct((NUM_KV_HEADS, S, HEAD_SIZE), jnp.bfloat16),
            jax.ShapeDtypeStruct((NUM_KV_HEADS, S, HEAD_SIZE), jnp.bfloat16),
            jax.ShapeDtypeStruct((NUM_KV_HEADS, S, HEAD_SIZE), jnp.bfloat16),
            jax.ShapeDtypeStruct((T, 2 * NUM_EXPERTS), jnp.float32),
        ],
    )(xf, xb, W_router, b_router.reshape(1, NUM_EXPERTS), W_noise,
      b_noise.reshape(1, NUM_EXPERTS), noise, win_b, wk_b, wv_b)

    o = pl.pallas_call(
        _attn_kernel,
        grid=(NUM_KV_HEADS, S // BQ),
        in_specs=[
            pl.BlockSpec((1, BQ, HEAD_SIZE), lambda h, i: (h, i, 0)),
            pl.BlockSpec((1, BQ, HEAD_SIZE), lambda h, i: (h, i, 0)),
            pl.BlockSpec((1, S, HEAD_SIZE), lambda h, i: (h, 0, 0)),
            pl.BlockSpec((1, S, HEAD_SIZE), lambda h, i: (h, 0, 0)),
        ],
        out_specs=pl.BlockSpec((TOP_K, 1, BQ, HEAD_SIZE), lambda h, i: (0, h, i, 0)),
        out_shape=jax.ShapeDtypeStruct((TOP_K, NUM_KV_HEADS, S, HEAD_SIZE),
                                       jnp.bfloat16),
        scratch_shapes=[pltpu.VMEM((2 * BQ, S), jnp.float32)],
    )(q0, q1, k, v)

    y = pl.pallas_call(
        _combine_kernel,
        grid=(T // BT,),
        in_specs=[
            pl.BlockSpec((1, NUM_KV_HEADS, BT, HEAD_SIZE), lambda i: (0, 0, i, 0)),
            pl.BlockSpec((1, NUM_KV_HEADS, BT, HEAD_SIZE), lambda i: (1, 0, i, 0)),
            pl.BlockSpec((BT, 2 * NUM_EXPERTS), lambda i: (i, 0)),
            pl.BlockSpec((NUM_EXPERTS, D, KV_PROJ), lambda i: (0, 0, 0)),
            pl.BlockSpec((1, D), lambda i: (0, 0)),
        ],
        out_specs=pl.BlockSpec((BT, D), lambda i: (i, 0)),
        out_shape=jax.ShapeDtypeStruct((T, D), jnp.float32),
    )(o, o, gm, wout_b, p_bias.reshape(1, D))

    return y.reshape(bsz, S, D)


# speedup vs baseline: 1.5920x; 1.0030x over previous
import jax
import jax.numpy as jnp
from jax.experimental import pallas as pl
from jax.experimental.pallas import tpu as pltpu

SEQ = 2048
NUM_HEADS = 16
HEAD_SIZE = 64
N_EMBED = 1024
NUM_EXPERTS = 8
TOP_K = 2
NUM_KV_HEADS = NUM_HEADS // TOP_K
KV_PROJ = NUM_KV_HEADS * HEAD_SIZE

BT = 512
BQ = 1024
BKV = 1024


def _dot_t(a, b):
    return jax.lax.dot_general(
        a, b, (((1,), (1,)), ((), ())), preferred_element_type=jnp.float32)


def _qkv_kernel(x_ref, xb_ref, wr_ref, br_ref, wn_ref, bn_ref, noise_ref,
                win_ref, wk_ref, wv_ref,
                q0_ref, q1_ref, k_ref, v_ref, gm_ref):
    x = x_ref[...]
    xb = xb_ref[...]
    logits = _dot_t(x, wr_ref[...]) + br_ref[...]
    nlog = _dot_t(x, wn_ref[...]) + bn_ref[...]
    noisy = logits + noise_ref[...] * jax.nn.softplus(nlog)

    lanes = jax.lax.broadcasted_iota(jnp.int32, noisy.shape, 1)
    i0 = jnp.argmax(noisy, axis=1)
    m0 = (lanes == i0[:, None])
    v0 = jnp.max(noisy, axis=1)
    masked = jnp.where(m0, -jnp.inf, noisy)
    i1 = jnp.argmax(masked, axis=1)
    v1 = jnp.max(masked, axis=1)
    m1 = (lanes == i1[:, None])
    e1 = jnp.exp(v1 - v0)
    g0 = 1.0 / (1.0 + e1)
    g1 = e1 / (1.0 + e1)

    m0f = m0.astype(jnp.float32)
    m1f = m1.astype(jnp.float32)
    gm0 = g0[:, None] * m0f
    gm1 = g1[:, None] * m1f
    gm_ref[...] = jnp.concatenate([gm0, gm1], axis=1)

    k_ref[...] = _dot_t(xb, wk_ref[...]).astype(jnp.bfloat16).reshape(
        -1, NUM_KV_HEADS, HEAD_SIZE).transpose(1, 0, 2)
    v_ref[...] = _dot_t(xb, wv_ref[...]).astype(jnp.bfloat16).reshape(
        -1, NUM_KV_HEADS, HEAD_SIZE).transpose(1, 0, 2)

    q0 = jnp.zeros((x.shape[0], KV_PROJ), jnp.float32)
    q1 = jnp.zeros((x.shape[0], KV_PROJ), jnp.float32)
    for e in range(NUM_EXPERTS):
        h = _dot_t(xb, win_ref[e])
        q0 = q0 + m0f[:, e:e + 1] * h
        q1 = q1 + m1f[:, e:e + 1] * h
    q0_ref[...] = q0.astype(jnp.bfloat16).reshape(
        -1, NUM_KV_HEADS, HEAD_SIZE).transpose(1, 0, 2)
    q1_ref[...] = q1.astype(jnp.bfloat16).reshape(
        -1, NUM_KV_HEADS, HEAD_SIZE).transpose(1, 0, 2)


def _attn_kernel(q0_ref, q1_ref, k_ref, v_ref, o_ref, s_scr):
    qi = pl.program_id(1)
    R = 2 * BQ
    q = jnp.concatenate([q0_ref[0], q1_ref[0]], axis=0)
    q = q * jnp.bfloat16(HEAD_SIZE ** -0.5)
    rows = qi * BQ + jax.lax.broadcasted_iota(jnp.int32, (R, BKV), 0) % BQ

    def fill(j, _):
        s_scr[:, pl.ds(j * BKV, BKV)] = jnp.full((R, BKV), -jnp.inf, jnp.float32)
        return 0

    def body(j, _):
        kb = k_ref[0, pl.ds(j * BKV, BKV), :]
        s = _dot_t(q, kb)
        cols = j * BKV + jax.lax.broadcasted_iota(jnp.int32, (R, BKV), 1)
        s_scr[:, pl.ds(j * BKV, BKV)] = jnp.where(cols <= rows, s, -jnp.inf)
        return 0

    nvalid = (qi + 1) * (BQ // BKV)
    jax.lax.fori_loop(nvalid, SEQ // BKV, fill, 0)
    jax.lax.fori_loop(0, nvalid, body, 0)

    sf = s_scr[...]
    mrow = jnp.max(sf, axis=1, keepdims=True)
    p = jnp.exp(sf - mrow)
    l = jnp.sum(p, axis=1, keepdims=True)
    av = jnp.dot(p.astype(jnp.bfloat16), v_ref[0],
                 preferred_element_type=jnp.float32)
    o_ref[...] = (av / l).astype(jnp.bfloat16).reshape(2, 1, BQ, HEAD_SIZE)


def _combine_kernel(ao0_ref, ao1_ref, gm_ref, wout_ref, bias_ref, y_ref):
    gm = gm_ref[...]
    bt = ao0_ref.shape[2]
    ao0 = ao0_ref[0].transpose(1, 0, 2).reshape(bt, KV_PROJ).astype(jnp.float32)
    ao1 = ao1_ref[0].transpose(1, 0, 2).reshape(bt, KV_PROJ).astype(jnp.float32)
    y = jnp.zeros((ao0.shape[0], N_EMBED), jnp.float32) + bias_ref[...]
    for e in range(NUM_EXPERTS):
        c = gm[:, e:e + 1] * ao0 + gm[:, NUM_EXPERTS + e:NUM_EXPERTS + e + 1] * ao1
        y = y + _dot_t(c.astype(jnp.bfloat16), wout_ref[e])
    y_ref[...] = y


@jax.jit
def kernel(x, W_router, b_router, W_noise, b_noise, W_in, W_out, W_k, W_v,
           p_bias, noise):
    bsz, S, D = x.shape
    T = bsz * S
    xf = x.reshape(T, D)
    xb = xf.astype(jnp.bfloat16)
    win_b = W_in.astype(jnp.bfloat16)
    wout_b = W_out.astype(jnp.bfloat16)
    wk_b = W_k.astype(jnp.bfloat16)
    wv_b = W_v.astype(jnp.bfloat16)

    q0, q1, k, v, gm = pl.pallas_call(
        _qkv_kernel,
        grid=(T // BT,),
        in_specs=[
            pl.BlockSpec((BT, D), lambda i: (i, 0)),
            pl.BlockSpec((BT, D), lambda i: (i, 0)),
            pl.BlockSpec((NUM_EXPERTS, D), lambda i: (0, 0)),
            pl.BlockSpec((1, NUM_EXPERTS), lambda i: (0, 0)),
            pl.BlockSpec((NUM_EXPERTS, D), lambda i: (0, 0)),
            pl.BlockSpec((1, NUM_EXPERTS), lambda i: (0, 0)),
            pl.BlockSpec((BT, NUM_EXPERTS), lambda i: (i, 0)),
            pl.BlockSpec((NUM_EXPERTS, KV_PROJ, D), lambda i: (0, 0, 0)),
            pl.BlockSpec((KV_PROJ, D), lambda i: (0, 0)),
            pl.BlockSpec((KV_PROJ, D), lambda i: (0, 0)),
        ],
        out_specs=[
            pl.BlockSpec((NUM_KV_HEADS, BT, HEAD_SIZE), lambda i: (0, i, 0)),
            pl.BlockSpec((NUM_KV_HEADS, BT, HEAD_SIZE), lambda i: (0, i, 0)),
            pl.BlockSpec((NUM_KV_HEADS, BT, HEAD_SIZE), lambda i: (0, i, 0)),
            pl.BlockSpec((NUM_KV_HEADS, BT, HEAD_SIZE), lambda i: (0, i, 0)),
            pl.BlockSpec((BT, 2 * NUM_EXPERTS), lambda i: (i, 0)),
        ],
        out_shape=[
            jax.ShapeDtypeStruct((NUM_KV_HEADS, S, HEAD_SIZE), jnp.bfloat16),
            jax.ShapeDtypeStruct((NUM_KV_HEADS, S, HEAD_SIZE), jnp.bfloat16),
            jax.ShapeDtypeStruct((NUM_KV_HEADS, S, HEAD_SIZE), jnp.bfloat16),
            jax.ShapeDtypeStruct((NUM_KV_HEADS, S, HEAD_SIZE), jnp.bfloat16),
            jax.ShapeDtypeStruct((T, 2 * NUM_EXPERTS), jnp.float32),
        ],
    )(xf, xb, W_router, b_router.reshape(1, NUM_EXPERTS), W_noise,
      b_noise.reshape(1, NUM_EXPERTS), noise, win_b, wk_b, wv_b)

    o = pl.pallas_call(
        _attn_kernel,
        grid=(NUM_KV_HEADS, S // BQ),
        in_specs=[
            pl.BlockSpec((1, BQ, HEAD_SIZE), lambda h, i: (h, i, 0)),
            pl.BlockSpec((1, BQ, HEAD_SIZE), lambda h, i: (h, i, 0)),
            pl.BlockSpec((1, S, HEAD_SIZE), lambda h, i: (h, 0, 0)),
            pl.BlockSpec((1, S, HEAD_SIZE), lambda h, i: (h, 0, 0)),
        ],
        out_specs=pl.BlockSpec((TOP_K, 1, BQ, HEAD_SIZE), lambda h, i: (0, h, i, 0)),
        out_shape=jax.ShapeDtypeStruct((TOP_K, NUM_KV_HEADS, S, HEAD_SIZE),
                                       jnp.bfloat16),
        scratch_shapes=[pltpu.VMEM((2 * BQ, S), jnp.float32)],
    )(q0, q1, k, v)

    y = pl.pallas_call(
        _combine_kernel,
        grid=(T // BT,),
        in_specs=[
            pl.BlockSpec((1, NUM_KV_HEADS, BT, HEAD_SIZE), lambda i: (0, 0, i, 0)),
            pl.BlockSpec((1, NUM_KV_HEADS, BT, HEAD_SIZE), lambda i: (1, 0, i, 0)),
            pl.BlockSpec((BT, 2 * NUM_EXPERTS), lambda i: (i, 0)),
            pl.BlockSpec((NUM_EXPERTS, D, KV_PROJ), lambda i: (0, 0, 0)),
            pl.BlockSpec((1, D), lambda i: (0, 0)),
        ],
        out_specs=pl.BlockSpec((BT, D), lambda i: (i, 0)),
        out_shape=jax.ShapeDtypeStruct((T, D), jnp.float32),
    )(o, o, gm, wout_b, p_bias.reshape(1, D))

    return y.reshape(bsz, S, D)
